# baseline (speedup 1.0000x reference)
import jax
import jax.numpy as jnp
from jax import lax
from jax.experimental import pallas as pl
from jax.experimental.pallas import tpu as pltpu
from jax.experimental.pallas import tpu_sc as plsc

N = 160000
D = 256
S = 10000

NC = 2
NS = 16
HALF = D // NC

S_PAD = 10240

ROWS_PER_TILE = N // NS
CHUNK = 80
NVEC = CHUNK // 16
NCHUNK = ROWS_PER_TILE // CHUNK
SEGS_PER_TILE = S_PAD // NS
SEG_CHUNK = 64
NSEG_CHUNK = SEGS_PER_TILE // SEG_CHUNK
DEN_GROUP = 128


_BLK = 2000


def _gate_body(x_ref, w_ref, b_ref, e_ref):
    g = jnp.dot(x_ref[...], w_ref[...], preferred_element_type=jnp.float32)
    e_ref[...] = jnp.exp(g + b_ref[0, 0])


def _gate_pass(x, w, b):
    grid = N // _BLK
    e2 = pl.pallas_call(
        _gate_body,
        grid=(grid,),
        in_specs=[
            pl.BlockSpec((_BLK, D), lambda i: (i, 0)),
            pl.BlockSpec((D, 1), lambda i: (0, 0)),
            pl.BlockSpec((1, 1), lambda i: (0, 0)),
        ],
        out_specs=pl.BlockSpec((_BLK, 1), lambda i: (i, 0)),
        out_shape=jax.ShapeDtypeStruct((N, 1), jnp.float32),
    )(x, w, b)
    return e2.reshape(N)


def _sc_body(x_hbm, e_hbm, seg_hbm, out_hbm,
             acc, dall, sbuf, ebuf, ibuf, dbuf, dloc, dsum):
    c = lax.axis_index("c")
    t = lax.axis_index("s")
    col0 = c * HALF
    zero16 = jnp.zeros((16,), jnp.float32)

    def _zero_row(j, _):
        for k in range(HALF // 16):
            dbuf[j, pl.ds(k * 16, 16)] = zero16
        return 0

    lax.fori_loop(0, SEG_CHUNK, _zero_row, 0)

    def _zero_dloc(j, _):
        dloc[pl.ds(j * 16, 16)] = zero16
        return 0

    lax.fori_loop(0, S_PAD // 16, _zero_dloc, 0)

    for i in range(NSEG_CHUNK):
        pltpu.sync_copy(dbuf, acc.at[pl.ds(t * SEGS_PER_TILE + i * SEG_CHUNK,
                                           SEG_CHUNK)])
    plsc.subcore_barrier()

    row0 = t * ROWS_PER_TILE

    def _chunk(i, _):
        r = row0 + i * CHUNK
        pltpu.sync_copy(x_hbm.at[pl.ds(r, CHUNK), pl.ds(col0, HALF)],
                        sbuf.at[:, pl.ds(0, HALF)])
        pltpu.sync_copy(e_hbm.at[pl.ds(r, CHUNK)], ebuf.at[pl.ds(0, CHUNK)])
        pltpu.sync_copy(seg_hbm.at[pl.ds(r, CHUNK)], ibuf)

        for v in range(NVEC):
            plsc.addupdate_scatter(dloc, [ibuf[pl.ds(v * 16, 16)]],
                                   ebuf[pl.ds(v * 16, 16)])

        def _row(j, _):
            ej = ebuf[pl.ds(j, 16)][0]
            for k in range(HALF // 16):
                v = sbuf[j, pl.ds(k * 16, 16)]
                sbuf[j, pl.ds(k * 16, 16)] = v * ej
            return 0

        lax.fori_loop(0, CHUNK, _row, 0)
        pltpu.sync_copy(sbuf, acc.at[ibuf], add=True)
        return 0

    lax.fori_loop(0, NCHUNK, _chunk, 0)
    pltpu.sync_copy(dloc, dall.at[pl.ds(t * S_PAD, S_PAD)])
    plsc.subcore_barrier()

    def _div_row(base):
        def _f(j, _):
            dj = dsum[pl.ds(base + j, 16)][0] + 1e-16
            for k in range(HALF // 16):
                dbuf[j, pl.ds(k * 16, 16)] = dbuf[j, pl.ds(k * 16, 16)] / dj
            return 0
        return _f

    for i in range(NSEG_CHUNK):
        seg0 = t * SEGS_PER_TILE + i * SEG_CHUNK
        if i % 2 == 0:
            for t2 in range(NS):
                pltpu.sync_copy(dall.at[pl.ds(t2 * S_PAD + seg0, DEN_GROUP)],
                                dsum.at[pl.ds(t2 * DEN_GROUP, DEN_GROUP)])
            for w in range(DEN_GROUP // 16):
                tot = dsum[pl.ds(w * 16, 16)]
                for t2 in range(1, NS):
                    tot = tot + dsum[pl.ds(t2 * DEN_GROUP + w * 16, 16)]
                dsum[pl.ds(w * 16, 16)] = tot
        pltpu.sync_copy(acc.at[pl.ds(seg0, SEG_CHUNK)], dbuf)
        lax.fori_loop(0, SEG_CHUNK, _div_row((i % 2) * SEG_CHUNK), 0)

        @pl.when(seg0 + SEG_CHUNK <= S)
        def _():
            pltpu.sync_copy(dbuf.at[:, pl.ds(0, HALF)],
                            out_hbm.at[pl.ds(seg0, SEG_CHUNK),
                                       pl.ds(col0, HALF)])

        @pl.when((seg0 < S) & (seg0 + SEG_CHUNK > S))
        def _():
            pltpu.sync_copy(dbuf.at[pl.ds(0, S % SEG_CHUNK), pl.ds(0, HALF)],
                            out_hbm.at[pl.ds(seg0, S % SEG_CHUNK),
                                       pl.ds(col0, HALF)])


_sc_pool = pl.kernel(
    _sc_body,
    out_type=jax.ShapeDtypeStruct((S, D), jnp.float32),
    mesh=plsc.VectorSubcoreMesh(core_axis_name="c", subcore_axis_name="s"),
    scratch_types=[
        pltpu.VMEM_SHARED((S_PAD, HALF), jnp.float32),
        pltpu.VMEM_SHARED((NS * S_PAD,), jnp.float32),
        pltpu.VMEM((CHUNK, HALF), jnp.float32),
        pltpu.VMEM((CHUNK + 16,), jnp.float32),
        pltpu.VMEM((CHUNK,), jnp.int32),
        pltpu.VMEM((SEG_CHUNK, HALF), jnp.float32),
        pltpu.VMEM((S_PAD,), jnp.float32),
        pltpu.VMEM((NS * DEN_GROUP + 16,), jnp.float32),
    ],
    compiler_params=pltpu.CompilerParams(needs_layout_passes=False),
)


@jax.jit
def kernel(x, batch, W, b):
    e = _gate_pass(x, W, b.reshape(1, 1))
    return _sc_pool(x, e, batch)


# speedup vs baseline: 1.3429x; 1.3429x over previous
import jax
import jax.numpy as jnp
from jax import lax
from jax.experimental import pallas as pl
from jax.experimental.pallas import tpu as pltpu
from jax.experimental.pallas import tpu_sc as plsc

N = 160000
D = 256
S = 10000

NC = 2
NS = 16
HALF = D // NC

S_PAD = 10240

ROWS_PER_TILE = N // NS
CHUNK = 80
NVEC = CHUNK // 16
NCHUNK = ROWS_PER_TILE // CHUNK
NPAIR = NCHUNK // 2
SEGS_PER_TILE = S_PAD // NS
SEG_CHUNK = 32
NSEG_CHUNK = SEGS_PER_TILE // SEG_CHUNK
DEN_GROUP = 128


_BLK = 2000


def _gate_body(x_ref, w_ref, b_ref, e_ref):
    g = jnp.dot(x_ref[...], w_ref[...], preferred_element_type=jnp.float32)
    e_ref[...] = jnp.exp(g + b_ref[0, 0])


def _gate_pass(x, w, b):
    grid = N // _BLK
    e2 = pl.pallas_call(
        _gate_body,
        grid=(grid,),
        in_specs=[
            pl.BlockSpec((_BLK, D), lambda i: (i, 0)),
            pl.BlockSpec((D, 1), lambda i: (0, 0)),
            pl.BlockSpec((1, 1), lambda i: (0, 0)),
        ],
        out_specs=pl.BlockSpec((_BLK, 1), lambda i: (i, 0)),
        out_shape=jax.ShapeDtypeStruct((N, 1), jnp.float32),
    )(x, w, b)
    return e2.reshape(N)


def _sc_body(x_hbm, e_hbm, seg_hbm, out_hbm, acc, dall,
             sbufa, ebufa, ibufa, sbufb, ebufb, ibufb,
             dbuf, dloc, dsum, sema, semb, semsa, semsb):
    c = lax.axis_index("c")
    t = lax.axis_index("s")
    col0 = c * HALF
    row0 = t * ROWS_PER_TILE
    zero16 = jnp.zeros((16,), jnp.float32)

    def fill(ci, sb, eb, ib, sem):
        r = row0 + ci * CHUNK
        pltpu.async_copy(x_hbm.at[pl.ds(r, CHUNK), pl.ds(col0, HALF)],
                         sb.at[:, pl.ds(0, HALF)], sem)
        pltpu.async_copy(e_hbm.at[pl.ds(r, CHUNK)], eb.at[pl.ds(0, CHUNK)], sem)
        pltpu.async_copy(seg_hbm.at[pl.ds(r, CHUNK)], ib, sem)

    def wait_fill(ci, sb, eb, ib, sem):
        r = row0 + ci * CHUNK
        pltpu.make_async_copy(x_hbm.at[pl.ds(r, CHUNK), pl.ds(col0, HALF)],
                              sb.at[:, pl.ds(0, HALF)], sem).wait()
        pltpu.make_async_copy(e_hbm.at[pl.ds(r, CHUNK)],
                              eb.at[pl.ds(0, CHUNK)], sem).wait()
        pltpu.make_async_copy(seg_hbm.at[pl.ds(r, CHUNK)], ib, sem).wait()

    def compute(sb, eb, ib):
        for v in range(NVEC):
            plsc.addupdate_scatter(dloc, [ib[pl.ds(v * 16, 16)]],
                                   eb[pl.ds(v * 16, 16)])

        def _row(j, _):
            ej = eb[pl.ds(j, 16)][0]
            for k in range(HALF // 16):
                sb[j, pl.ds(k * 16, 16)] = sb[j, pl.ds(k * 16, 16)] * ej
            return 0

        lax.fori_loop(0, CHUNK, _row, 0)

    def scat(sb, ib, sem):
        pltpu.async_copy(sb, acc.at[ib], sem, add=True)

    def wait_scat(sb, ib, sem):
        pltpu.make_async_copy(sb, acc.at[ib], sem).wait()

    fill(0, sbufa, ebufa, ibufa, sema)

    def _zero_row(j, _):
        for k in range(HALF // 16):
            dbuf[j, pl.ds(k * 16, 16)] = zero16
        return 0

    lax.fori_loop(0, SEG_CHUNK, _zero_row, 0)

    def _zero_dloc(j, _):
        dloc[pl.ds(j * 16, 16)] = zero16
        return 0

    lax.fori_loop(0, S_PAD // 16, _zero_dloc, 0)

    for i in range(NSEG_CHUNK):
        pltpu.sync_copy(dbuf, acc.at[pl.ds(t * SEGS_PER_TILE + i * SEG_CHUNK,
                                           SEG_CHUNK)])
    plsc.subcore_barrier()

    def _pair(k, _):
        a = 2 * k
        wait_fill(a, sbufa, ebufa, ibufa, sema)
        compute(sbufa, ebufa, ibufa)

        @pl.when(k > 0)
        def _():
            wait_scat(sbufb, ibufb, semsb)

        fill(a + 1, sbufb, ebufb, ibufb, semb)
        scat(sbufa, ibufa, semsa)
        wait_fill(a + 1, sbufb, ebufb, ibufb, semb)
        compute(sbufb, ebufb, ibufb)
        wait_scat(sbufa, ibufa, semsa)
        fill(a + 2, sbufa, ebufa, ibufa, sema)
        scat(sbufb, ibufb, semsb)
        return 0

    lax.fori_loop(0, NPAIR, _pair, 0)

    wait_fill(NCHUNK - 1, sbufa, ebufa, ibufa, sema)
    compute(sbufa, ebufa, ibufa)
    wait_scat(sbufb, ibufb, semsb)
    scat(sbufa, ibufa, semsa)
    wait_scat(sbufa, ibufa, semsa)

    pltpu.sync_copy(dloc, dall.at[pl.ds(t * S_PAD, S_PAD)])
    plsc.subcore_barrier()

    def _div_row(base):
        def _f(j, _):
            dj = dsum[pl.ds(base + j, 16)][0] + 1e-16
            for k in range(HALF // 16):
                dbuf[j, pl.ds(k * 16, 16)] = dbuf[j, pl.ds(k * 16, 16)] / dj
            return 0
        return _f

    for i in range(NSEG_CHUNK):
        seg0 = t * SEGS_PER_TILE + i * SEG_CHUNK
        if i % 4 == 0:
            for t2 in range(NS):
                pltpu.sync_copy(dall.at[pl.ds(t2 * S_PAD + seg0, DEN_GROUP)],
                                dsum.at[pl.ds(t2 * DEN_GROUP, DEN_GROUP)])
            for w in range(DEN_GROUP // 16):
                tot = dsum[pl.ds(w * 16, 16)]
                for t2 in range(1, NS):
                    tot = tot + dsum[pl.ds(t2 * DEN_GROUP + w * 16, 16)]
                dsum[pl.ds(w * 16, 16)] = tot
        pltpu.sync_copy(acc.at[pl.ds(seg0, SEG_CHUNK)], dbuf)
        lax.fori_loop(0, SEG_CHUNK, _div_row((i % 4) * SEG_CHUNK), 0)

        @pl.when(seg0 + SEG_CHUNK <= S)
        def _():
            pltpu.sync_copy(dbuf.at[:, pl.ds(0, HALF)],
                            out_hbm.at[pl.ds(seg0, SEG_CHUNK),
                                       pl.ds(col0, HALF)])

        @pl.when((seg0 < S) & (seg0 + SEG_CHUNK > S))
        def _():
            pltpu.sync_copy(dbuf.at[pl.ds(0, S % SEG_CHUNK), pl.ds(0, HALF)],
                            out_hbm.at[pl.ds(seg0, S % SEG_CHUNK),
                                       pl.ds(col0, HALF)])


_sc_pool = pl.kernel(
    _sc_body,
    out_type=jax.ShapeDtypeStruct((S, D), jnp.float32),
    mesh=plsc.VectorSubcoreMesh(core_axis_name="c", subcore_axis_name="s"),
    scratch_types=[
        pltpu.VMEM_SHARED((S_PAD, HALF), jnp.float32),
        pltpu.VMEM_SHARED((NS * S_PAD,), jnp.float32),
        pltpu.VMEM((CHUNK, HALF), jnp.float32),
        pltpu.VMEM((CHUNK + 16,), jnp.float32),
        pltpu.VMEM((CHUNK,), jnp.int32),
        pltpu.VMEM((CHUNK, HALF), jnp.float32),
        pltpu.VMEM((CHUNK + 16,), jnp.float32),
        pltpu.VMEM((CHUNK,), jnp.int32),
        pltpu.VMEM((SEG_CHUNK, HALF), jnp.float32),
        pltpu.VMEM((S_PAD,), jnp.float32),
        pltpu.VMEM((NS * DEN_GROUP + 16,), jnp.float32),
        pltpu.SemaphoreType.DMA,
        pltpu.SemaphoreType.DMA,
        pltpu.SemaphoreType.DMA,
        pltpu.SemaphoreType.DMA,
    ],
    compiler_params=pltpu.CompilerParams(needs_layout_passes=False),
)


@jax.jit
def kernel(x, batch, W, b):
    e = _gate_pass(x, W, b.reshape(1, 1))
    return _sc_pool(x, e, batch)


# speedup vs baseline: 1.4861x; 1.1066x over previous
import jax
import jax.numpy as jnp
from jax import lax
from jax.experimental import pallas as pl
from jax.experimental.pallas import tpu as pltpu
from jax.experimental.pallas import tpu_sc as plsc

N = 160000
D = 256
S = 10000

NC = 2
NS = 16
HALF = D // NC

S_PAD = 10240

ROWS_PER_TILE = N // NS
CHUNK = 80
NVEC = CHUNK // 16
NCHUNK = ROWS_PER_TILE // CHUNK
NPAIR = NCHUNK // 2
SEGS_PER_TILE = S_PAD // NS
SEG_CHUNK = 32
NSEG_CHUNK = SEGS_PER_TILE // SEG_CHUNK
DEN_GROUP = 128


_BLK = 8000


def _gate_body(x_ref, w_ref, b_ref, e_ref):
    g = jnp.dot(x_ref[...], w_ref[...], preferred_element_type=jnp.float32)
    e_ref[...] = jnp.exp(g + b_ref[0, 0])


def _gate_pass(x, w, b):
    grid = N // _BLK
    e2 = pl.pallas_call(
        _gate_body,
        grid=(grid,),
        in_specs=[
            pl.BlockSpec((_BLK, D), lambda i: (i, 0)),
            pl.BlockSpec((D, 1), lambda i: (0, 0)),
            pl.BlockSpec((1, 1), lambda i: (0, 0)),
        ],
        out_specs=pl.BlockSpec((_BLK, 1), lambda i: (i, 0)),
        out_shape=jax.ShapeDtypeStruct((N, 1), jnp.float32),
    )(x, w, b)
    return e2.reshape(N)


def _sc_body(x_hbm, e_hbm, seg_hbm, out_hbm, acc, dall,
             sbufa, ebufa, ibufa, sbufb, ebufb, ibufb,
             dbuf, dloc, dsum, sema, semb, semsa, semsb):
    c = lax.axis_index("c")
    t = lax.axis_index("s")
    col0 = c * HALF
    row0 = t * ROWS_PER_TILE
    zero16 = jnp.zeros((16,), jnp.float32)

    def fill(ci, sb, eb, ib, sem):
        r = row0 + ci * CHUNK
        pltpu.async_copy(x_hbm.at[pl.ds(r, CHUNK), pl.ds(col0, HALF)],
                         sb.at[:, pl.ds(0, HALF)], sem)
        pltpu.async_copy(e_hbm.at[pl.ds(r, CHUNK)], eb.at[pl.ds(0, CHUNK)], sem)
        pltpu.async_copy(seg_hbm.at[pl.ds(r, CHUNK)], ib, sem)

    def wait_fill(ci, sb, eb, ib, sem):
        r = row0 + ci * CHUNK
        pltpu.make_async_copy(x_hbm.at[pl.ds(r, CHUNK), pl.ds(col0, HALF)],
                              sb.at[:, pl.ds(0, HALF)], sem).wait()
        pltpu.make_async_copy(e_hbm.at[pl.ds(r, CHUNK)],
                              eb.at[pl.ds(0, CHUNK)], sem).wait()
        pltpu.make_async_copy(seg_hbm.at[pl.ds(r, CHUNK)], ib, sem).wait()

    def compute(sb, eb, ib):
        for v in range(NVEC):
            plsc.addupdate_scatter(dloc, [ib[pl.ds(v * 16, 16)]],
                                   eb[pl.ds(v * 16, 16)])

        def _row(j, _):
            ej = eb[pl.ds(j, 16)][0]
            for k in range(HALF // 16):
                sb[j, pl.ds(k * 16, 16)] = sb[j, pl.ds(k * 16, 16)] * ej
            return 0

        lax.fori_loop(0, CHUNK, _row, 0, unroll=4)

    def scat(sb, ib, sem):
        pltpu.async_copy(sb, acc.at[ib], sem, add=True)

    def wait_scat(sb, ib, sem):
        pltpu.make_async_copy(sb, acc.at[ib], sem).wait()

    fill(0, sbufa, ebufa, ibufa, sema)

    def _zero_row(j, _):
        for k in range(HALF // 16):
            dbuf[j, pl.ds(k * 16, 16)] = zero16
        return 0

    lax.fori_loop(0, SEG_CHUNK, _zero_row, 0)

    def _zero_dloc(j, _):
        dloc[pl.ds(j * 16, 16)] = zero16
        return 0

    lax.fori_loop(0, S_PAD // 16, _zero_dloc, 0)

    for i in range(NSEG_CHUNK):
        pltpu.sync_copy(dbuf, acc.at[pl.ds(t * SEGS_PER_TILE + i * SEG_CHUNK,
                                           SEG_CHUNK)])
    plsc.subcore_barrier()

    def _pair(k, _):
        a = 2 * k
        wait_fill(a, sbufa, ebufa, ibufa, sema)
        compute(sbufa, ebufa, ibufa)

        @pl.when(k > 0)
        def _():
            wait_scat(sbufb, ibufb, semsb)

        fill(a + 1, sbufb, ebufb, ibufb, semb)
        scat(sbufa, ibufa, semsa)
        wait_fill(a + 1, sbufb, ebufb, ibufb, semb)
        compute(sbufb, ebufb, ibufb)
        wait_scat(sbufa, ibufa, semsa)
        fill(a + 2, sbufa, ebufa, ibufa, sema)
        scat(sbufb, ibufb, semsb)
        return 0

    lax.fori_loop(0, NPAIR, _pair, 0)

    wait_fill(NCHUNK - 1, sbufa, ebufa, ibufa, sema)
    compute(sbufa, ebufa, ibufa)
    wait_scat(sbufb, ibufb, semsb)
    scat(sbufa, ibufa, semsa)
    wait_scat(sbufa, ibufa, semsa)

    pltpu.sync_copy(dloc, dall.at[pl.ds(t * S_PAD, S_PAD)])
    plsc.subcore_barrier()

    def _div_row(base):
        def _f(j, _):
            dj = dsum[pl.ds(base + j, 16)][0] + 1e-16
            for k in range(HALF // 16):
                dbuf[j, pl.ds(k * 16, 16)] = dbuf[j, pl.ds(k * 16, 16)] / dj
            return 0
        return _f

    for i in range(NSEG_CHUNK):
        seg0 = t * SEGS_PER_TILE + i * SEG_CHUNK
        if i % 4 == 0:
            for t2 in range(NS):
                pltpu.sync_copy(dall.at[pl.ds(t2 * S_PAD + seg0, DEN_GROUP)],
                                dsum.at[pl.ds(t2 * DEN_GROUP, DEN_GROUP)])
            for w in range(DEN_GROUP // 16):
                tot = dsum[pl.ds(w * 16, 16)]
                for t2 in range(1, NS):
                    tot = tot + dsum[pl.ds(t2 * DEN_GROUP + w * 16, 16)]
                dsum[pl.ds(w * 16, 16)] = tot
        pltpu.sync_copy(acc.at[pl.ds(seg0, SEG_CHUNK)], dbuf)
        lax.fori_loop(0, SEG_CHUNK, _div_row((i % 4) * SEG_CHUNK), 0)

        @pl.when(seg0 + SEG_CHUNK <= S)
        def _():
            pltpu.sync_copy(dbuf.at[:, pl.ds(0, HALF)],
                            out_hbm.at[pl.ds(seg0, SEG_CHUNK),
                                       pl.ds(col0, HALF)])

        @pl.when((seg0 < S) & (seg0 + SEG_CHUNK > S))
        def _():
            pltpu.sync_copy(dbuf.at[pl.ds(0, S % SEG_CHUNK), pl.ds(0, HALF)],
                            out_hbm.at[pl.ds(seg0, S % SEG_CHUNK),
                                       pl.ds(col0, HALF)])


_sc_pool = pl.kernel(
    _sc_body,
    out_type=jax.ShapeDtypeStruct((S, D), jnp.float32),
    mesh=plsc.VectorSubcoreMesh(core_axis_name="c", subcore_axis_name="s"),
    scratch_types=[
        pltpu.VMEM_SHARED((S_PAD, HALF), jnp.float32),
        pltpu.VMEM_SHARED((NS * S_PAD,), jnp.float32),
        pltpu.VMEM((CHUNK, HALF), jnp.float32),
        pltpu.VMEM((CHUNK + 16,), jnp.float32),
        pltpu.VMEM((CHUNK,), jnp.int32),
        pltpu.VMEM((CHUNK, HALF), jnp.float32),
        pltpu.VMEM((CHUNK + 16,), jnp.float32),
        pltpu.VMEM((CHUNK,), jnp.int32),
        pltpu.VMEM((SEG_CHUNK, HALF), jnp.float32),
        pltpu.VMEM((S_PAD,), jnp.float32),
        pltpu.VMEM((NS * DEN_GROUP + 16,), jnp.float32),
        pltpu.SemaphoreType.DMA,
        pltpu.SemaphoreType.DMA,
        pltpu.SemaphoreType.DMA,
        pltpu.SemaphoreType.DMA,
    ],
    compiler_params=pltpu.CompilerParams(needs_layout_passes=False),
)


@jax.jit
def kernel(x, batch, W, b):
    e = _gate_pass(x, W, b.reshape(1, 1))
    return _sc_pool(x, e, batch)


# speedup vs baseline: 1.8876x; 1.2701x over previous
import jax
import jax.numpy as jnp
from jax import lax
from jax.experimental import pallas as pl
from jax.experimental.pallas import tpu as pltpu
from jax.experimental.pallas import tpu_sc as plsc

N = 160000
D = 256
S = 10000

NC = 2
NS = 16
HALF = D // NC

S_PAD = 10240

CHUNK = 64
NVEC = CHUNK // 16
CHUNKS_HI = 157
CHUNKS_LO = 156
HI_TILES = 4
NBODY = CHUNKS_LO // 3
SEGS_PER_TILE = S_PAD // NS
SEG_CHUNK = 16
NSEG_CHUNK = SEGS_PER_TILE // SEG_CHUNK
DEN_GROUP = 64


_BLK = 8000


def _gate_body(x_ref, w_ref, b_ref, e_ref):
    g = jnp.dot(x_ref[...], w_ref[...], preferred_element_type=jnp.float32)
    e_ref[...] = jnp.exp(g + b_ref[0, 0])


def _gate_pass(x, w, b):
    grid = N // _BLK
    e2 = pl.pallas_call(
        _gate_body,
        grid=(grid,),
        in_specs=[
            pl.BlockSpec((_BLK, D), lambda i: (i, 0)),
            pl.BlockSpec((D, 1), lambda i: (0, 0)),
            pl.BlockSpec((1, 1), lambda i: (0, 0)),
        ],
        out_specs=pl.BlockSpec((_BLK, 1), lambda i: (i, 0)),
        out_shape=jax.ShapeDtypeStruct((N, 1), jnp.float32),
    )(x, w, b)
    return e2.reshape(N)


def _sc_body(x_hbm, e_hbm, seg_hbm, out_hbm, acc, dall,
             sbufa, ebufa, ibufa, sbufb, ebufb, ibufb, sbufc, ebufc, ibufc,
             dbuf, dloc, dsum,
             sema, semb, semc, semsa, semsb, semsc):
    c = lax.axis_index("c")
    t = lax.axis_index("s")
    col0 = c * HALF
    nch = jnp.where(t < HI_TILES, CHUNKS_HI, CHUNKS_LO)
    row0 = CHUNK * (CHUNKS_HI * jnp.minimum(t, HI_TILES)
                    + CHUNKS_LO * jnp.maximum(t - HI_TILES, 0))
    zero16 = jnp.zeros((16,), jnp.float32)

    bufs = ((sbufa, ebufa, ibufa, sema, semsa),
            (sbufb, ebufb, ibufb, semb, semsb),
            (sbufc, ebufc, ibufc, semc, semsc))

    def fill(ci, sb, eb, ib, sem):
        r = row0 + ci * CHUNK
        pltpu.async_copy(x_hbm.at[pl.ds(r, CHUNK), pl.ds(col0, HALF)],
                         sb.at[:, pl.ds(0, HALF)], sem)
        pltpu.async_copy(e_hbm.at[pl.ds(r, CHUNK)], eb.at[pl.ds(0, CHUNK)], sem)
        pltpu.async_copy(seg_hbm.at[pl.ds(r, CHUNK)], ib, sem)

    def wait_fill(ci, sb, eb, ib, sem):
        r = row0 + ci * CHUNK
        pltpu.make_async_copy(x_hbm.at[pl.ds(r, CHUNK), pl.ds(col0, HALF)],
                              sb.at[:, pl.ds(0, HALF)], sem).wait()
        pltpu.make_async_copy(e_hbm.at[pl.ds(r, CHUNK)],
                              eb.at[pl.ds(0, CHUNK)], sem).wait()
        pltpu.make_async_copy(seg_hbm.at[pl.ds(r, CHUNK)], ib, sem).wait()

    def compute(sb, eb, ib):
        for v in range(NVEC):
            plsc.addupdate_scatter(dloc, [ib[pl.ds(v * 16, 16)]],
                                   eb[pl.ds(v * 16, 16)])

        def _row(j, _):
            ej = eb[pl.ds(j, 16)][0]
            for k in range(HALF // 16):
                sb[j, pl.ds(k * 16, 16)] = sb[j, pl.ds(k * 16, 16)] * ej
            return 0

        lax.fori_loop(0, CHUNK, _row, 0, unroll=4)

    def scat(sb, ib, sem):
        pltpu.async_copy(sb, acc.at[ib], sem, add=True)

    def wait_scat(sb, ib, sem):
        pltpu.make_async_copy(sb, acc.at[ib], sem).wait()

    fill(0, sbufa, ebufa, ibufa, sema)

    def _zero_row(j, _):
        for k in range(HALF // 16):
            dbuf[j, pl.ds(k * 16, 16)] = zero16
        return 0

    lax.fori_loop(0, SEG_CHUNK, _zero_row, 0)

    def _zero_dloc(j, _):
        dloc[pl.ds(j * 16, 16)] = zero16
        return 0

    lax.fori_loop(0, S_PAD // 16, _zero_dloc, 0)

    for i in range(NSEG_CHUNK):
        pltpu.sync_copy(dbuf, acc.at[pl.ds(t * SEGS_PER_TILE + i * SEG_CHUNK,
                                           SEG_CHUNK)])
    plsc.subcore_barrier()

    def _body(k, _):
        for off in range(3):
            ci = 3 * k + off
            sb, eb, ib, sem, ssem = bufs[off]
            nb, neb, nib, nsem, nssem = bufs[(off + 1) % 3]

            if off == 2:
                wait_scat(nb, nib, nssem)
            else:
                @pl.when(k > 0)
                def _():
                    wait_scat(nb, nib, nssem)

            @pl.when(ci + 1 < nch)
            def _():
                fill(ci + 1, nb, neb, nib, nsem)

            wait_fill(ci, sb, eb, ib, sem)
            compute(sb, eb, ib)
            scat(sb, ib, ssem)
        return 0

    lax.fori_loop(0, NBODY, _body, 0)

    @pl.when(nch == CHUNKS_HI)
    def _():
        wait_fill(CHUNKS_HI - 1, sbufa, ebufa, ibufa, sema)
        compute(sbufa, ebufa, ibufa)
        scat(sbufa, ibufa, semsa)
        wait_scat(sbufa, ibufa, semsa)

    wait_scat(sbufb, ibufb, semsb)
    wait_scat(sbufc, ibufc, semsc)

    pltpu.sync_copy(dloc, dall.at[pl.ds(t * S_PAD, S_PAD)])
    plsc.subcore_barrier()

    def _div_row(base):
        def _f(j, _):
            dj = dsum[pl.ds(base + j, 16)][0] + 1e-16
            for k in range(HALF // 16):
                dbuf[j, pl.ds(k * 16, 16)] = dbuf[j, pl.ds(k * 16, 16)] / dj
            return 0
        return _f

    GROUP_CHUNKS = DEN_GROUP // SEG_CHUNK
    for i in range(NSEG_CHUNK):
        seg0 = t * SEGS_PER_TILE + i * SEG_CHUNK
        if i % GROUP_CHUNKS == 0:
            for t2 in range(NS):
                pltpu.sync_copy(dall.at[pl.ds(t2 * S_PAD + seg0, DEN_GROUP)],
                                dsum.at[pl.ds(t2 * DEN_GROUP, DEN_GROUP)])
            for w in range(DEN_GROUP // 16):
                tot = dsum[pl.ds(w * 16, 16)]
                for t2 in range(1, NS):
                    tot = tot + dsum[pl.ds(t2 * DEN_GROUP + w * 16, 16)]
                dsum[pl.ds(w * 16, 16)] = tot
        pltpu.sync_copy(acc.at[pl.ds(seg0, SEG_CHUNK)], dbuf)
        lax.fori_loop(0, SEG_CHUNK, _div_row((i % GROUP_CHUNKS) * SEG_CHUNK),
                      0)

        @pl.when(seg0 + SEG_CHUNK <= S)
        def _():
            pltpu.sync_copy(dbuf.at[:, pl.ds(0, HALF)],
                            out_hbm.at[pl.ds(seg0, SEG_CHUNK),
                                       pl.ds(col0, HALF)])


_sc_pool = pl.kernel(
    _sc_body,
    out_type=jax.ShapeDtypeStruct((S, D), jnp.float32),
    mesh=plsc.VectorSubcoreMesh(core_axis_name="c", subcore_axis_name="s"),
    scratch_types=[
        pltpu.VMEM_SHARED((S_PAD, HALF), jnp.float32),
        pltpu.VMEM_SHARED((NS * S_PAD,), jnp.float32),
        pltpu.VMEM((CHUNK, HALF), jnp.float32),
        pltpu.VMEM((CHUNK + 16,), jnp.float32),
        pltpu.VMEM((CHUNK,), jnp.int32),
        pltpu.VMEM((CHUNK, HALF), jnp.float32),
        pltpu.VMEM((CHUNK + 16,), jnp.float32),
        pltpu.VMEM((CHUNK,), jnp.int32),
        pltpu.VMEM((CHUNK, HALF), jnp.float32),
        pltpu.VMEM((CHUNK + 16,), jnp.float32),
        pltpu.VMEM((CHUNK,), jnp.int32),
        pltpu.VMEM((SEG_CHUNK, HALF), jnp.float32),
        pltpu.VMEM((S_PAD,), jnp.float32),
        pltpu.VMEM((NS * DEN_GROUP + 16,), jnp.float32),
        pltpu.SemaphoreType.DMA,
        pltpu.SemaphoreType.DMA,
        pltpu.SemaphoreType.DMA,
        pltpu.SemaphoreType.DMA,
        pltpu.SemaphoreType.DMA,
        pltpu.SemaphoreType.DMA,
    ],
    compiler_params=pltpu.CompilerParams(needs_layout_passes=False),
)


@jax.jit
def kernel(x, batch, W, b):
    e = _gate_pass(x, W, b.reshape(1, 1))
    return _sc_pool(x, e, batch)
